# Initial kernel scaffold; baseline (speedup 1.0000x reference)
import jax
import jax.numpy as jnp
from jax.experimental import pallas as pl


def kernel(x, edge_index, W1, att_src1, att_dst1, b1, W2, att_src2, att_dst2, b2):
    raise NotImplementedError("write your pallas kernel here")


# speedup vs baseline: 40.2736x; 40.2736x over previous
import functools

import jax
import jax.numpy as jnp
from jax import lax
from jax.experimental import pallas as pl
from jax.experimental.pallas import tpu as pltpu
from jax.experimental.pallas import tpu_sc as plsc

NEG = -1e9
_SC_PARAMS = pltpu.CompilerParams(needs_layout_passes=False)
BR = 512
BLK = 128
NW = 32
LANES = 16


def _round_up(a, b):
    return (a + b - 1) // b * b


def _tc1(x_pad, W1, att12, n_valid):
    np_, d = x_pad.shape

    def body(x_ref, w_ref, a12_ref, h_ref, a_ref):
        i = pl.program_id(0)
        h = lax.dot_general(x_ref[...], w_ref[...], (((1,), (0,)), ((), ())),
                            preferred_element_type=jnp.float32,
                            precision=lax.Precision.HIGHEST)
        h_ref[...] = h
        a = lax.dot_general(a12_ref[...], h, (((1,), (1,)), ((), ())),
                            preferred_element_type=jnp.float32,
                            precision=lax.Precision.HIGHEST)
        col = i * BR + lax.broadcasted_iota(jnp.int32, (2, BR), 1)
        a = jnp.where(col < n_valid, a, NEG)
        a_ref[...] = jnp.concatenate([a, jnp.zeros((6, BR), jnp.float32)], 0)

    return pl.pallas_call(
        body,
        grid=(np_ // BR,),
        in_specs=[
            pl.BlockSpec((BR, d), lambda i: (i, 0)),
            pl.BlockSpec((d, d), lambda i: (0, 0)),
            pl.BlockSpec((2, d), lambda i: (0, 0)),
        ],
        out_specs=[
            pl.BlockSpec((BR, d), lambda i: (i, 0)),
            pl.BlockSpec((8, BR), lambda i: (0, i)),
        ],
        out_shape=[
            jax.ShapeDtypeStruct((np_, d), jnp.float32),
            jax.ShapeDtypeStruct((8, np_), jnp.float32),
        ],
    )(x_pad, W1, att12)


def _sc1(h, a1, src, dst, nb):
    np_, d = h.shape
    npt = np_ // 16
    nchunk = BLK // LANES
    mesh = plsc.VectorSubcoreMesh(core_axis_name="c", subcore_axis_name="s")

    @functools.partial(
        pl.kernel,
        out_type=[
            jax.ShapeDtypeStruct((2, np_, d), jnp.float32),
            jax.ShapeDtypeStruct((2, 8, np_), jnp.float32),
        ],
        mesh=mesh,
        compiler_params=_SC_PARAMS,
        scratch_types=[
            pltpu.VMEM((np_,), jnp.float32),
            pltpu.VMEM((np_,), jnp.float32),
            pltpu.VMEM((BLK,), jnp.int32),
            pltpu.VMEM((BLK,), jnp.int32),
            pltpu.VMEM((BLK,), jnp.float32),
            pltpu.VMEM((BLK, d), jnp.float32),
            pltpu.VMEM((npt,), jnp.float32),
            pltpu.VMEM_SHARED((np_, d), jnp.float32),
            pltpu.VMEM_SHARED((np_,), jnp.float32),
        ],
    )
    def k(h_hbm, a1_hbm, src_hbm, dst_hbm, u_hbm, d_hbm,
          asrc_l, adst_l, sidx, didx, pvec, msg, zbuf, u_sh, d_sh):
        cid = lax.axis_index("c")
        sid = lax.axis_index("s")
        wid = cid * 16 + sid

        pltpu.sync_copy(a1_hbm.at[0], asrc_l)
        pltpu.sync_copy(a1_hbm.at[1], adst_l)

        @pl.loop(0, BLK)
        def _(r):
            for c in range(nchunk):
                msg[r, pl.ds(c * LANES, LANES)] = jnp.zeros((LANES,), jnp.float32)

        @pl.loop(0, npt, step=LANES)
        def _(i):
            zbuf[pl.ds(i, LANES)] = jnp.zeros((LANES,), jnp.float32)

        for kk in range(npt // BLK):
            pltpu.sync_copy(msg, u_sh.at[pl.ds(sid * npt + kk * BLK, BLK)])
        pltpu.sync_copy(zbuf, d_sh.at[pl.ds(sid * npt, npt)])
        plsc.subcore_barrier()

        base_w = wid * (nb * BLK)

        @pl.loop(0, nb)
        def _(b):
            base = pl.multiple_of(base_w + b * BLK, BLK)
            pltpu.sync_copy(src_hbm.at[pl.ds(base, BLK)], sidx)
            pltpu.sync_copy(dst_hbm.at[pl.ds(base, BLK)], didx)
            pltpu.sync_copy(h_hbm.at[sidx], msg)
            for c in range(nchunk):
                sv = sidx[pl.ds(c * LANES, LANES)]
                dv = didx[pl.ds(c * LANES, LANES)]
                e = (plsc.load_gather(asrc_l, [sv])
                     + plsc.load_gather(adst_l, [dv]))
                e = jnp.where(e >= 0.0, e, 0.2 * e)
                pvec[pl.ds(c * LANES, LANES)] = jnp.exp(e)
            pltpu.sync_copy(pvec, d_sh.at[didx], add=True)

            @pl.loop(0, BLK, step=LANES)
            def _(rb):
                pv = pvec[pl.ds(rb, LANES)]
                for j in range(LANES):
                    pr = pv[j]
                    for c in range(nchunk):
                        sl = pl.ds(c * LANES, LANES)
                        msg[rb + j, sl] = msg[rb + j, sl] * pr

            pltpu.sync_copy(msg, u_sh.at[didx], add=True)

        plsc.subcore_barrier()
        pltpu.sync_copy(u_sh.at[pl.ds(sid * npt, npt)],
                        u_hbm.at[cid, pl.ds(sid * npt, npt)])
        pltpu.sync_copy(d_sh.at[pl.ds(sid * npt, npt)],
                        d_hbm.at[cid, 0, pl.ds(sid * npt, npt)])

    return k(h, a1, src, dst)


def _tc2(u1p, d1p, b1, W2T, att_s2, att_d2, n_valid):
    _, np_, d = u1p.shape

    def body(u_ref, dn_ref, b1_ref, w2t_ref, as2_ref, ad2_ref, b2_ref):
        i = pl.program_id(0)
        u = u_ref[0] + u_ref[1]
        den = dn_ref[0, 0:1, :] + dn_ref[1, 0:1, :]
        denc = jnp.transpose(den, (1, 0))
        h1 = jnp.maximum(u / (denc + 1e-16) + b1_ref[...], 0.0)
        c01 = lax.dot_general(w2t_ref[...], h1, (((1,), (1,)), ((), ())),
                              preferred_element_type=jnp.float32,
                              precision=lax.Precision.HIGHEST)
        as2 = as2_ref[0] * c01[0:1, :] + as2_ref[1] * c01[1:2, :]
        ad2 = ad2_ref[0] * c01[0:1, :] + ad2_ref[1] * c01[1:2, :]
        col = i * BR + lax.broadcasted_iota(jnp.int32, (1, BR), 1)
        as2 = jnp.where(col < n_valid, as2, NEG)
        ad2 = jnp.where(col < n_valid, ad2, NEG)
        b2_ref[...] = jnp.concatenate(
            [c01, as2, ad2, jnp.zeros((4, BR), jnp.float32)], 0)

    return pl.pallas_call(
        body,
        grid=(np_ // BR,),
        in_specs=[
            pl.BlockSpec((2, BR, d), lambda i: (0, i, 0)),
            pl.BlockSpec((2, 8, BR), lambda i: (0, 0, i)),
            pl.BlockSpec((1, d), lambda i: (0, 0)),
            pl.BlockSpec((2, d), lambda i: (0, 0)),
            pl.BlockSpec(memory_space=pltpu.SMEM),
            pl.BlockSpec(memory_space=pltpu.SMEM),
        ],
        out_specs=pl.BlockSpec((8, BR), lambda i: (0, i)),
        out_shape=jax.ShapeDtypeStruct((8, np_), jnp.float32),
    )(u1p, d1p, b1, W2T, att_s2, att_d2)


def _sc2(b2arr, src, dst, nb):
    _, np_ = b2arr.shape
    npt = np_ // 16
    nchunk = BLK // LANES
    mesh = plsc.VectorSubcoreMesh(core_axis_name="c", subcore_axis_name="s")

    @functools.partial(
        pl.kernel,
        out_type=jax.ShapeDtypeStruct((2, 8, np_), jnp.float32),
        mesh=mesh,
        compiler_params=_SC_PARAMS,
        scratch_types=[
            pltpu.VMEM((np_,), jnp.float32),
            pltpu.VMEM((np_,), jnp.float32),
            pltpu.VMEM((np_,), jnp.float32),
            pltpu.VMEM((np_,), jnp.float32),
            pltpu.VMEM((BLK,), jnp.int32),
            pltpu.VMEM((BLK,), jnp.int32),
            pltpu.VMEM((BLK,), jnp.float32),
            pltpu.VMEM((BLK,), jnp.float32),
            pltpu.VMEM((BLK,), jnp.float32),
            pltpu.VMEM((npt,), jnp.float32),
            pltpu.VMEM_SHARED((np_,), jnp.float32),
            pltpu.VMEM_SHARED((np_,), jnp.float32),
            pltpu.VMEM_SHARED((np_,), jnp.float32),
        ],
    )
    def k(b2_hbm, src_hbm, dst_hbm, o_hbm,
          h2a_l, h2b_l, as2_l, ad2_l, sidx, didx, pa, pb, pc, zbuf,
          ua_sh, ub_sh, d2_sh):
        cid = lax.axis_index("c")
        sid = lax.axis_index("s")
        wid = cid * 16 + sid

        pltpu.sync_copy(b2_hbm.at[0], h2a_l)
        pltpu.sync_copy(b2_hbm.at[1], h2b_l)
        pltpu.sync_copy(b2_hbm.at[2], as2_l)
        pltpu.sync_copy(b2_hbm.at[3], ad2_l)

        @pl.loop(0, npt, step=LANES)
        def _(i):
            zbuf[pl.ds(i, LANES)] = jnp.zeros((LANES,), jnp.float32)

        sl_out = pl.ds(sid * npt, npt)
        pltpu.sync_copy(zbuf, ua_sh.at[sl_out])
        pltpu.sync_copy(zbuf, ub_sh.at[sl_out])
        pltpu.sync_copy(zbuf, d2_sh.at[sl_out])
        plsc.subcore_barrier()

        base_w = wid * (nb * BLK)

        @pl.loop(0, nb)
        def _(b):
            base = pl.multiple_of(base_w + b * BLK, BLK)
            pltpu.sync_copy(src_hbm.at[pl.ds(base, BLK)], sidx)
            pltpu.sync_copy(dst_hbm.at[pl.ds(base, BLK)], didx)
            for c in range(nchunk):
                sl = pl.ds(c * LANES, LANES)
                sv = sidx[sl]
                dv = didx[sl]
                e = (plsc.load_gather(as2_l, [sv])
                     + plsc.load_gather(ad2_l, [dv]))
                e = jnp.where(e >= 0.0, e, 0.2 * e)
                p = jnp.exp(e)
                pa[sl] = p * plsc.load_gather(h2a_l, [sv])
                pb[sl] = p * plsc.load_gather(h2b_l, [sv])
                pc[sl] = p
            pltpu.sync_copy(pa, ua_sh.at[didx], add=True)
            pltpu.sync_copy(pb, ub_sh.at[didx], add=True)
            pltpu.sync_copy(pc, d2_sh.at[didx], add=True)

        plsc.subcore_barrier()
        pltpu.sync_copy(ua_sh.at[sl_out], o_hbm.at[cid, 0, sl_out])
        pltpu.sync_copy(ub_sh.at[sl_out], o_hbm.at[cid, 1, sl_out])
        pltpu.sync_copy(d2_sh.at[sl_out], o_hbm.at[cid, 2, sl_out])

    return k(b2arr, src, dst)


def _tc3(o2p, b2, n):
    _, _, np_ = o2p.shape

    def body(o_ref, b2_ref, out_ref):
        u0 = o_ref[0, 0:1, :] + o_ref[1, 0:1, :]
        u1 = o_ref[0, 1:2, :] + o_ref[1, 1:2, :]
        dd = o_ref[0, 2:3, :] + o_ref[1, 2:3, :] + 1e-16
        o0 = u0 / dd + b2_ref[0]
        o1 = u1 / dd + b2_ref[1]
        out_ref[...] = jnp.transpose(jnp.concatenate([o0, o1], 0), (1, 0))

    return pl.pallas_call(
        body,
        grid=(np_ // BR,),
        in_specs=[
            pl.BlockSpec((2, 8, BR), lambda i: (0, 0, i)),
            pl.BlockSpec(memory_space=pltpu.SMEM),
        ],
        out_specs=pl.BlockSpec((BR, 2), lambda i: (i, 0)),
        out_shape=jax.ShapeDtypeStruct((n, 2), jnp.float32),
    )(o2p, b2)


def kernel(x, edge_index, W1, att_src1, att_dst1, b1, W2, att_src2,
           att_dst2, b2):
    n, d = x.shape
    e = edge_index.shape[1]
    n_pad = _round_up(n, BR)
    e_tot = e + n
    nb = _round_up(e_tot, NW * BLK) // (NW * BLK)
    e_pad = NW * nb * BLK
    n_fill = e_pad - e_tot

    ei = edge_index.astype(jnp.int32)
    loop_idx = jnp.arange(n, dtype=jnp.int32)
    fill = jnp.arange(n_fill, dtype=jnp.int32)
    pad_src = (fill * 131) % n
    pad_dst = n + fill % (n_pad - n)
    src = jnp.concatenate([ei[0], loop_idx, pad_src])
    dst = jnp.concatenate([ei[1], loop_idx, pad_dst])

    x_pad = jnp.pad(x, ((0, n_pad - n), (0, 0)))
    att12 = jnp.stack([att_src1, att_dst1])
    b1r = b1.reshape(1, d)
    w2t = jnp.transpose(W2)

    h, a1 = _tc1(x_pad, W1, att12, n)
    u1p, d1p = _sc1(h, a1, src, dst, nb)
    b2arr = _tc2(u1p, d1p, b1r, w2t, att_src2, att_dst2, n)
    o2p = _sc2(b2arr, src, dst, nb)
    return _tc3(o2p, b2, n)


# speedup vs baseline: 57.0591x; 1.4168x over previous
import functools

import jax
import jax.numpy as jnp
from jax import lax
from jax.experimental import pallas as pl
from jax.experimental.pallas import tpu as pltpu
from jax.experimental.pallas import tpu_sc as plsc

NEG = -1e9
_SC_PARAMS = pltpu.CompilerParams(needs_layout_passes=False)
BR = 512
BLK = 128
NW = 32
LANES = 16


def _round_up(a, b):
    return (a + b - 1) // b * b


def _tc1(x_pad, W1, att12, n_valid):
    np_, d = x_pad.shape

    def body(x_ref, w_ref, a12_ref, h_ref, a_ref):
        i = pl.program_id(0)
        h = lax.dot_general(x_ref[...], w_ref[...], (((1,), (0,)), ((), ())),
                            preferred_element_type=jnp.float32,
                            precision=lax.Precision.HIGHEST)
        h_ref[...] = h
        a = lax.dot_general(a12_ref[...], h, (((1,), (1,)), ((), ())),
                            preferred_element_type=jnp.float32,
                            precision=lax.Precision.HIGHEST)
        col = i * BR + lax.broadcasted_iota(jnp.int32, (2, BR), 1)
        a = jnp.where(col < n_valid, a, NEG)
        a_ref[...] = jnp.concatenate([a, jnp.zeros((6, BR), jnp.float32)], 0)

    return pl.pallas_call(
        body,
        grid=(np_ // BR,),
        in_specs=[
            pl.BlockSpec((BR, d), lambda i: (i, 0)),
            pl.BlockSpec((d, d), lambda i: (0, 0)),
            pl.BlockSpec((2, d), lambda i: (0, 0)),
        ],
        out_specs=[
            pl.BlockSpec((BR, d), lambda i: (i, 0)),
            pl.BlockSpec((8, BR), lambda i: (0, i)),
        ],
        out_shape=[
            jax.ShapeDtypeStruct((np_, d), jnp.float32),
            jax.ShapeDtypeStruct((8, np_), jnp.float32),
        ],
    )(x_pad, W1, att12)


def _sc1(h, a1, edges, nb):
    np_, d = h.shape
    npt = np_ // 16
    nchunk = BLK // LANES
    mesh = plsc.VectorSubcoreMesh(core_axis_name="c", subcore_axis_name="s")

    @functools.partial(
        pl.kernel,
        out_type=[
            jax.ShapeDtypeStruct((2, np_, d), jnp.float32),
            jax.ShapeDtypeStruct((2, 8, np_), jnp.float32),
        ],
        mesh=mesh,
        compiler_params=_SC_PARAMS,
        scratch_types=[
            pltpu.VMEM((2, 2, BLK), jnp.int32),
            pltpu.VMEM((2, BLK), jnp.float32),
            pltpu.VMEM((2, BLK), jnp.float32),
            pltpu.VMEM((2, BLK), jnp.float32),
            pltpu.VMEM((2, BLK, d), jnp.float32),
            pltpu.VMEM((npt,), jnp.float32),
            pltpu.VMEM_SHARED((np_, d), jnp.float32),
            pltpu.VMEM_SHARED((np_,), jnp.float32),
            pltpu.VMEM_SHARED((np_,), jnp.float32),
            pltpu.VMEM_SHARED((np_,), jnp.float32),
            pltpu.SemaphoreType.DMA,
            pltpu.SemaphoreType.DMA,
            pltpu.SemaphoreType.DMA,
            pltpu.SemaphoreType.DMA,
            pltpu.SemaphoreType.DMA,
            pltpu.SemaphoreType.DMA,
        ],
    )
    def k(h_hbm, a1_hbm, e_hbm, u_hbm, d_hbm,
          eidx2, pvec2, av2, bv2, msg2, zbuf, u_sh, d_sh,
          asrc_sh, adst_sh, gsem0, gsem1, ssem0, ssem1, asem0, asem1):
        cid = lax.axis_index("c")
        sid = lax.axis_index("s")
        wid = cid * 16 + sid
        gsem = (gsem0, gsem1)
        ssem = (ssem0, ssem1)
        asem = (asem0, asem1)

        sl_npt = pl.ds(sid * npt, npt)
        pltpu.sync_copy(a1_hbm.at[0, sl_npt], asrc_sh.at[sl_npt])
        pltpu.sync_copy(a1_hbm.at[1, sl_npt], adst_sh.at[sl_npt])

        msg = msg2.at[0]

        @pl.loop(0, BLK)
        def _(r):
            for c in range(nchunk):
                msg[r, pl.ds(c * LANES, LANES)] = jnp.zeros((LANES,), jnp.float32)

        @pl.loop(0, npt, step=LANES)
        def _(i):
            zbuf[pl.ds(i, LANES)] = jnp.zeros((LANES,), jnp.float32)

        for kk in range(npt // BLK):
            pltpu.sync_copy(msg, u_sh.at[pl.ds(sid * npt + kk * BLK, BLK)])
        pltpu.sync_copy(zbuf, d_sh.at[pl.ds(sid * npt, npt)])
        plsc.subcore_barrier()

        blk0 = wid * nb

        @pl.loop(0, nb, step=2)
        def _(b):
            for x in range(2):
                bb = b + x
                sidx = eidx2.at[x, 0]
                didx = eidx2.at[x, 1]
                pvec = pvec2.at[x]
                av = av2.at[x]
                bv = bv2.at[x]
                msg = msg2.at[x]

                @pl.when(bb >= 2)
                def _():
                    pltpu.make_async_copy(msg, u_sh.at[didx], ssem[x]).wait()

                pltpu.sync_copy(e_hbm.at[blk0 + bb], eidx2.at[x])
                gat = pltpu.async_copy(h_hbm.at[sidx], msg, gsem[x])
                ga = pltpu.async_copy(asrc_sh.at[sidx], av, asem[x])
                gb = pltpu.async_copy(adst_sh.at[didx], bv, asem[x])
                ga.wait()
                gb.wait()
                for c in range(nchunk):
                    sl = pl.ds(c * LANES, LANES)
                    e = av[sl] + bv[sl]
                    e = jnp.where(e >= 0.0, e, 0.2 * e)
                    pvec[sl] = jnp.exp(e)
                pltpu.sync_copy(pvec, d_sh.at[didx], add=True)
                gat.wait()

                @pl.loop(0, BLK, step=LANES)
                def _(rb):
                    pv = pvec[pl.ds(rb, LANES)]
                    for j in range(LANES):
                        pr = pv[j]
                        for c in range(nchunk):
                            sl = pl.ds(c * LANES, LANES)
                            msg[rb + j, sl] = msg[rb + j, sl] * pr

                pltpu.async_copy(msg, u_sh.at[didx], ssem[x], add=True)

        for x in range(2):
            pltpu.make_async_copy(msg2.at[x], u_sh.at[eidx2.at[x, 1]],
                                  ssem[x]).wait()
        plsc.subcore_barrier()
        pltpu.sync_copy(u_sh.at[pl.ds(sid * npt, npt)],
                        u_hbm.at[cid, pl.ds(sid * npt, npt)])
        pltpu.sync_copy(d_sh.at[pl.ds(sid * npt, npt)],
                        d_hbm.at[cid, 0, pl.ds(sid * npt, npt)])

    return k(h, a1, edges)


def _tc2(u1p, d1p, b1, W2T, att_s2, att_d2, n_valid):
    _, np_, d = u1p.shape

    def body(u_ref, dn_ref, b1_ref, w2t_ref, as2_ref, ad2_ref, b2_ref):
        i = pl.program_id(0)
        u = u_ref[0] + u_ref[1]
        den = dn_ref[0, 0:1, :] + dn_ref[1, 0:1, :]
        denc = jnp.transpose(den, (1, 0))
        h1 = jnp.maximum(u / (denc + 1e-16) + b1_ref[...], 0.0)
        c01 = lax.dot_general(w2t_ref[...], h1, (((1,), (1,)), ((), ())),
                              preferred_element_type=jnp.float32,
                              precision=lax.Precision.HIGHEST)
        as2 = as2_ref[0] * c01[0:1, :] + as2_ref[1] * c01[1:2, :]
        ad2 = ad2_ref[0] * c01[0:1, :] + ad2_ref[1] * c01[1:2, :]
        col = i * BR + lax.broadcasted_iota(jnp.int32, (1, BR), 1)
        as2 = jnp.where(col < n_valid, as2, NEG)
        ad2 = jnp.where(col < n_valid, ad2, NEG)
        b2_ref[...] = jnp.concatenate(
            [c01, as2, ad2, jnp.zeros((4, BR), jnp.float32)], 0)

    return pl.pallas_call(
        body,
        grid=(np_ // BR,),
        in_specs=[
            pl.BlockSpec((2, BR, d), lambda i: (0, i, 0)),
            pl.BlockSpec((2, 8, BR), lambda i: (0, 0, i)),
            pl.BlockSpec((1, d), lambda i: (0, 0)),
            pl.BlockSpec((2, d), lambda i: (0, 0)),
            pl.BlockSpec(memory_space=pltpu.SMEM),
            pl.BlockSpec(memory_space=pltpu.SMEM),
        ],
        out_specs=pl.BlockSpec((8, BR), lambda i: (0, i)),
        out_shape=jax.ShapeDtypeStruct((8, np_), jnp.float32),
    )(u1p, d1p, b1, W2T, att_s2, att_d2)


def _sc2(b2arr, edges, nb):
    _, np_ = b2arr.shape
    npt = np_ // 16
    nchunk = BLK // LANES
    mesh = plsc.VectorSubcoreMesh(core_axis_name="c", subcore_axis_name="s")

    @functools.partial(
        pl.kernel,
        out_type=jax.ShapeDtypeStruct((2, 8, np_), jnp.float32),
        mesh=mesh,
        compiler_params=_SC_PARAMS,
        scratch_types=[
            pltpu.VMEM((np_,), jnp.float32),
            pltpu.VMEM((np_,), jnp.float32),
            pltpu.VMEM((np_,), jnp.float32),
            pltpu.VMEM((np_,), jnp.float32),
            pltpu.VMEM((2, 2, BLK), jnp.int32),
            pltpu.VMEM((2, BLK), jnp.float32),
            pltpu.VMEM((2, BLK), jnp.float32),
            pltpu.VMEM((2, BLK), jnp.float32),
            pltpu.VMEM((npt,), jnp.float32),
            pltpu.VMEM_SHARED((np_,), jnp.float32),
            pltpu.VMEM_SHARED((np_,), jnp.float32),
            pltpu.VMEM_SHARED((np_,), jnp.float32),
            pltpu.SemaphoreType.DMA,
            pltpu.SemaphoreType.DMA,
        ],
    )
    def k(b2_hbm, e_hbm, o_hbm,
          h2a_l, h2b_l, as2_l, ad2_l, eidx2, pa2, pb2, pc2, zbuf,
          ua_sh, ub_sh, d2_sh, ssem0, ssem1):
        cid = lax.axis_index("c")
        sid = lax.axis_index("s")
        wid = cid * 16 + sid
        ssem = (ssem0, ssem1)

        pltpu.sync_copy(b2_hbm.at[0], h2a_l)
        pltpu.sync_copy(b2_hbm.at[1], h2b_l)
        pltpu.sync_copy(b2_hbm.at[2], as2_l)
        pltpu.sync_copy(b2_hbm.at[3], ad2_l)

        @pl.loop(0, npt, step=LANES)
        def _(i):
            zbuf[pl.ds(i, LANES)] = jnp.zeros((LANES,), jnp.float32)

        sl_out = pl.ds(sid * npt, npt)
        pltpu.sync_copy(zbuf, ua_sh.at[sl_out])
        pltpu.sync_copy(zbuf, ub_sh.at[sl_out])
        pltpu.sync_copy(zbuf, d2_sh.at[sl_out])
        plsc.subcore_barrier()

        blk0 = wid * nb

        @pl.loop(0, nb, step=2)
        def _(b):
            for x in range(2):
                bb = b + x
                sidx = eidx2.at[x, 0]
                didx = eidx2.at[x, 1]
                pa = pa2.at[x]
                pb = pb2.at[x]
                pc = pc2.at[x]

                @pl.when(bb >= 2)
                def _():
                    pltpu.make_async_copy(pa, ua_sh.at[didx], ssem[x]).wait()
                    pltpu.make_async_copy(pb, ub_sh.at[didx], ssem[x]).wait()
                    pltpu.make_async_copy(pc, d2_sh.at[didx], ssem[x]).wait()

                pltpu.sync_copy(e_hbm.at[blk0 + bb], eidx2.at[x])
                for c in range(nchunk):
                    sl = pl.ds(c * LANES, LANES)
                    sv = sidx[sl]
                    dv = didx[sl]
                    e = (plsc.load_gather(as2_l, [sv])
                         + plsc.load_gather(ad2_l, [dv]))
                    e = jnp.where(e >= 0.0, e, 0.2 * e)
                    p = jnp.exp(e)
                    pa[sl] = p * plsc.load_gather(h2a_l, [sv])
                    pb[sl] = p * plsc.load_gather(h2b_l, [sv])
                    pc[sl] = p
                pltpu.async_copy(pa, ua_sh.at[didx], ssem[x], add=True)
                pltpu.async_copy(pb, ub_sh.at[didx], ssem[x], add=True)
                pltpu.async_copy(pc, d2_sh.at[didx], ssem[x], add=True)

        for x in range(2):
            pltpu.make_async_copy(pa2.at[x], ua_sh.at[eidx2.at[x, 1]],
                                  ssem[x]).wait()
            pltpu.make_async_copy(pb2.at[x], ub_sh.at[eidx2.at[x, 1]],
                                  ssem[x]).wait()
            pltpu.make_async_copy(pc2.at[x], d2_sh.at[eidx2.at[x, 1]],
                                  ssem[x]).wait()
        plsc.subcore_barrier()
        pltpu.sync_copy(ua_sh.at[sl_out], o_hbm.at[cid, 0, sl_out])
        pltpu.sync_copy(ub_sh.at[sl_out], o_hbm.at[cid, 1, sl_out])
        pltpu.sync_copy(d2_sh.at[sl_out], o_hbm.at[cid, 2, sl_out])

    return k(b2arr, edges)


def _tc3(o2p, b2, n):
    _, _, np_ = o2p.shape

    def body(o_ref, b2_ref, out_ref):
        u0 = o_ref[0, 0:1, :] + o_ref[1, 0:1, :]
        u1 = o_ref[0, 1:2, :] + o_ref[1, 1:2, :]
        dd = o_ref[0, 2:3, :] + o_ref[1, 2:3, :] + 1e-16
        o0 = u0 / dd + b2_ref[0]
        o1 = u1 / dd + b2_ref[1]
        out_ref[...] = jnp.transpose(jnp.concatenate([o0, o1], 0), (1, 0))

    return pl.pallas_call(
        body,
        grid=(np_ // BR,),
        in_specs=[
            pl.BlockSpec((2, 8, BR), lambda i: (0, 0, i)),
            pl.BlockSpec(memory_space=pltpu.SMEM),
        ],
        out_specs=pl.BlockSpec((BR, 2), lambda i: (i, 0)),
        out_shape=jax.ShapeDtypeStruct((n, 2), jnp.float32),
    )(o2p, b2)


def kernel(x, edge_index, W1, att_src1, att_dst1, b1, W2, att_src2,
           att_dst2, b2):
    n, d = x.shape
    e = edge_index.shape[1]
    n_pad = _round_up(n, BR)
    e_tot = e + n
    nb = _round_up(e_tot, NW * BLK) // (NW * BLK)
    nb = nb + (nb % 2)
    e_pad = NW * nb * BLK
    n_fill = e_pad - e_tot

    ei = edge_index.astype(jnp.int32)
    loop_idx = jnp.arange(n, dtype=jnp.int32)
    fill = jnp.arange(n_fill, dtype=jnp.int32)
    pad_src = (fill * 131) % n
    pad_dst = n + fill % (n_pad - n)
    src = jnp.concatenate([ei[0], loop_idx, pad_src])
    dst = jnp.concatenate([ei[1], loop_idx, pad_dst])
    edges = jnp.stack([src.reshape(-1, BLK), dst.reshape(-1, BLK)], axis=1)

    x_pad = jnp.pad(x, ((0, n_pad - n), (0, 0)))
    att12 = jnp.stack([att_src1, att_dst1])
    b1r = b1.reshape(1, d)
    w2t = jnp.transpose(W2)

    h, a1 = _tc1(x_pad, W1, att12, n)
    u1p, d1p = _sc1(h, a1, edges, nb)
    b2arr = _tc2(u1p, d1p, b1r, w2t, att_src2, att_dst2, n)
    o2p = _sc2(b2arr, edges, nb)
    return _tc3(o2p, b2, n)


# speedup vs baseline: 65.9143x; 1.1552x over previous
import functools

import jax
import jax.numpy as jnp
from jax import lax
from jax.experimental import pallas as pl
from jax.experimental.pallas import tpu as pltpu
from jax.experimental.pallas import tpu_sc as plsc

NEG = -1e9
_SC_PARAMS = pltpu.CompilerParams(needs_layout_passes=False)
BR = 512
BLK = 128
NW = 32
LANES = 16


def _round_up(a, b):
    return (a + b - 1) // b * b


def _tc1(x_pad, W1, att12, n_valid):
    np_, d = x_pad.shape

    def body(x_ref, w_ref, a12_ref, h_ref, a_ref):
        i = pl.program_id(0)
        h = lax.dot_general(x_ref[...], w_ref[...], (((1,), (0,)), ((), ())),
                            preferred_element_type=jnp.float32,
                            precision=lax.Precision.HIGHEST)
        h_ref[...] = h
        a = lax.dot_general(a12_ref[...], h, (((1,), (1,)), ((), ())),
                            preferred_element_type=jnp.float32,
                            precision=lax.Precision.HIGHEST)
        col = i * BR + lax.broadcasted_iota(jnp.int32, (2, BR), 1)
        a = jnp.where(col < n_valid, a, NEG)
        a_ref[...] = jnp.concatenate([a, jnp.zeros((6, BR), jnp.float32)], 0)

    return pl.pallas_call(
        body,
        grid=(np_ // BR,),
        in_specs=[
            pl.BlockSpec((BR, d), lambda i: (i, 0)),
            pl.BlockSpec((d, d), lambda i: (0, 0)),
            pl.BlockSpec((2, d), lambda i: (0, 0)),
        ],
        out_specs=[
            pl.BlockSpec((BR, d), lambda i: (i, 0)),
            pl.BlockSpec((8, BR), lambda i: (0, i)),
        ],
        out_shape=[
            jax.ShapeDtypeStruct((np_, d), jnp.float32),
            jax.ShapeDtypeStruct((8, np_), jnp.float32),
        ],
    )(x_pad, W1, att12)


def _sc1(h, a1, edges, nb):
    np_, d = h.shape
    npt = np_ // 16
    nchunk = BLK // LANES
    mesh = plsc.VectorSubcoreMesh(core_axis_name="c", subcore_axis_name="s")

    @functools.partial(
        pl.kernel,
        out_type=[
            jax.ShapeDtypeStruct((2, np_, d), jnp.float32),
            jax.ShapeDtypeStruct((2, 8, np_), jnp.float32),
        ],
        mesh=mesh,
        compiler_params=_SC_PARAMS,
        scratch_types=[
            pltpu.VMEM((3, 2, BLK), jnp.int32),
            pltpu.VMEM((2, BLK), jnp.float32),
            pltpu.VMEM((BLK,), jnp.float32),
            pltpu.VMEM((BLK,), jnp.float32),
            pltpu.VMEM((2, BLK, d), jnp.float32),
            pltpu.VMEM((npt,), jnp.float32),
            pltpu.VMEM_SHARED((np_, d), jnp.float32),
            pltpu.VMEM_SHARED((np_,), jnp.float32),
            pltpu.VMEM_SHARED((np_,), jnp.float32),
            pltpu.VMEM_SHARED((np_,), jnp.float32),
            pltpu.SemaphoreType.DMA,
            pltpu.SemaphoreType.DMA,
            pltpu.SemaphoreType.DMA,
            pltpu.SemaphoreType.DMA,
            pltpu.SemaphoreType.DMA,
            pltpu.SemaphoreType.DMA,
            pltpu.SemaphoreType.DMA,
            pltpu.SemaphoreType.DMA,
            pltpu.SemaphoreType.DMA,
        ],
    )
    def k(h_hbm, a1_hbm, e_hbm, u_hbm, d_hbm,
          eidx3, pvec2, av, bv, msg2, zbuf, u_sh, d_sh,
          asrc_sh, adst_sh, gsem0, gsem1, ssem0, ssem1, psem0, psem1,
          isem0, isem1, asem):
        cid = lax.axis_index("c")
        sid = lax.axis_index("s")
        wid = cid * 16 + sid
        gsem = (gsem0, gsem1)
        ssem = (ssem0, ssem1)
        psem = (psem0, psem1)
        isem = (isem0, isem1)

        sl_npt = pl.ds(sid * npt, npt)
        pltpu.sync_copy(a1_hbm.at[0, sl_npt], asrc_sh.at[sl_npt])
        pltpu.sync_copy(a1_hbm.at[1, sl_npt], adst_sh.at[sl_npt])

        msg = msg2.at[0]

        @pl.loop(0, BLK)
        def _(r):
            for c in range(nchunk):
                msg[r, pl.ds(c * LANES, LANES)] = jnp.zeros((LANES,), jnp.float32)

        @pl.loop(0, npt, step=LANES)
        def _(i):
            zbuf[pl.ds(i, LANES)] = jnp.zeros((LANES,), jnp.float32)

        for kk in range(npt // BLK):
            pltpu.sync_copy(msg, u_sh.at[pl.ds(sid * npt + kk * BLK, BLK)])
        pltpu.sync_copy(zbuf, d_sh.at[pl.ds(sid * npt, npt)])
        plsc.subcore_barrier()

        blk0 = wid * nb

        pltpu.async_copy(e_hbm.at[blk0], eidx3.at[0], isem0)

        @pl.loop(0, nb, step=6)
        def _(b):
            for u in range(6):
                bb = b + u
                x2 = u % 2
                x3 = u % 3
                sidx = eidx3.at[x3, 0]
                didx = eidx3.at[x3, 1]
                didx_o = eidx3.at[(u + 1) % 3, 1]
                pvec = pvec2.at[x2]
                msg = msg2.at[x2]

                def _drain():
                    pltpu.make_async_copy(msg, u_sh.at[didx_o],
                                          ssem[x2]).wait()
                    pltpu.make_async_copy(pvec, d_sh.at[didx_o],
                                          psem[x2]).wait()
                if u < 2:
                    @pl.when(bb >= 2)
                    def _():
                        _drain()
                else:
                    _drain()

                def _pref():
                    pltpu.async_copy(e_hbm.at[blk0 + bb + 1],
                                     eidx3.at[(u + 1) % 3],
                                     isem[(u + 1) % 2])
                if u == 5:
                    @pl.when(bb + 1 < nb)
                    def _():
                        _pref()
                else:
                    _pref()

                pltpu.make_async_copy(e_hbm.at[blk0 + bb], eidx3.at[x3],
                                      isem[x2]).wait()

                gat = pltpu.async_copy(h_hbm.at[sidx], msg, gsem[x2])
                ga = pltpu.async_copy(asrc_sh.at[sidx], av, asem)
                gb = pltpu.async_copy(adst_sh.at[didx], bv, asem)
                ga.wait()
                gb.wait()
                for c in range(nchunk):
                    sl = pl.ds(c * LANES, LANES)
                    e = av[sl] + bv[sl]
                    e = jnp.where(e >= 0.0, e, 0.2 * e)
                    pvec[sl] = jnp.exp(e)
                pltpu.async_copy(pvec, d_sh.at[didx], psem[x2], add=True)
                gat.wait()

                @pl.loop(0, BLK, step=LANES)
                def _(rb):
                    pv = pvec[pl.ds(rb, LANES)]
                    for j in range(LANES):
                        pr = pv[j]
                        for c in range(nchunk):
                            sl = pl.ds(c * LANES, LANES)
                            msg[rb + j, sl] = msg[rb + j, sl] * pr

                pltpu.async_copy(msg, u_sh.at[didx], ssem[x2], add=True)

        for x2, x3 in ((0, 1), (1, 2)):
            pltpu.make_async_copy(msg2.at[x2], u_sh.at[eidx3.at[x3, 1]],
                                  ssem[x2]).wait()
            pltpu.make_async_copy(pvec2.at[x2], d_sh.at[eidx3.at[x3, 1]],
                                  psem[x2]).wait()
        plsc.subcore_barrier()
        pltpu.sync_copy(u_sh.at[pl.ds(sid * npt, npt)],
                        u_hbm.at[cid, pl.ds(sid * npt, npt)])
        pltpu.sync_copy(d_sh.at[pl.ds(sid * npt, npt)],
                        d_hbm.at[cid, 0, pl.ds(sid * npt, npt)])

    return k(h, a1, edges)


def _tc2(u1p, d1p, b1, W2T, att_s2, att_d2, n_valid):
    _, np_, d = u1p.shape

    def body(u_ref, dn_ref, b1_ref, w2t_ref, as2_ref, ad2_ref, b2_ref):
        i = pl.program_id(0)
        u = u_ref[0] + u_ref[1]
        den = dn_ref[0, 0:1, :] + dn_ref[1, 0:1, :]
        denc = jnp.transpose(den, (1, 0))
        h1 = jnp.maximum(u / (denc + 1e-16) + b1_ref[...], 0.0)
        c01 = lax.dot_general(w2t_ref[...], h1, (((1,), (1,)), ((), ())),
                              preferred_element_type=jnp.float32,
                              precision=lax.Precision.HIGHEST)
        as2 = as2_ref[0] * c01[0:1, :] + as2_ref[1] * c01[1:2, :]
        ad2 = ad2_ref[0] * c01[0:1, :] + ad2_ref[1] * c01[1:2, :]
        col = i * BR + lax.broadcasted_iota(jnp.int32, (1, BR), 1)
        as2 = jnp.where(col < n_valid, as2, NEG)
        ad2 = jnp.where(col < n_valid, ad2, NEG)
        b2_ref[...] = jnp.concatenate(
            [c01, as2, ad2, jnp.zeros((4, BR), jnp.float32)], 0)

    return pl.pallas_call(
        body,
        grid=(np_ // BR,),
        in_specs=[
            pl.BlockSpec((2, BR, d), lambda i: (0, i, 0)),
            pl.BlockSpec((2, 8, BR), lambda i: (0, 0, i)),
            pl.BlockSpec((1, d), lambda i: (0, 0)),
            pl.BlockSpec((2, d), lambda i: (0, 0)),
            pl.BlockSpec(memory_space=pltpu.SMEM),
            pl.BlockSpec(memory_space=pltpu.SMEM),
        ],
        out_specs=pl.BlockSpec((8, BR), lambda i: (0, i)),
        out_shape=jax.ShapeDtypeStruct((8, np_), jnp.float32),
    )(u1p, d1p, b1, W2T, att_s2, att_d2)


def _sc2(b2arr, edges, nb):
    _, np_ = b2arr.shape
    npt = np_ // 16
    nchunk = BLK // LANES
    mesh = plsc.VectorSubcoreMesh(core_axis_name="c", subcore_axis_name="s")

    @functools.partial(
        pl.kernel,
        out_type=jax.ShapeDtypeStruct((2, 8, np_), jnp.float32),
        mesh=mesh,
        compiler_params=_SC_PARAMS,
        scratch_types=[
            pltpu.VMEM((np_,), jnp.float32),
            pltpu.VMEM((np_,), jnp.float32),
            pltpu.VMEM((np_,), jnp.float32),
            pltpu.VMEM((np_,), jnp.float32),
            pltpu.VMEM((3, 2, BLK), jnp.int32),
            pltpu.VMEM((2, BLK), jnp.float32),
            pltpu.VMEM((2, BLK), jnp.float32),
            pltpu.VMEM((2, BLK), jnp.float32),
            pltpu.VMEM((npt,), jnp.float32),
            pltpu.VMEM_SHARED((np_,), jnp.float32),
            pltpu.VMEM_SHARED((np_,), jnp.float32),
            pltpu.VMEM_SHARED((np_,), jnp.float32),
            pltpu.SemaphoreType.DMA,
            pltpu.SemaphoreType.DMA,
            pltpu.SemaphoreType.DMA,
            pltpu.SemaphoreType.DMA,
        ],
    )
    def k(b2_hbm, e_hbm, o_hbm,
          h2a_l, h2b_l, as2_l, ad2_l, eidx3, pa2, pb2, pc2, zbuf,
          ua_sh, ub_sh, d2_sh, ssem0, ssem1, isem0, isem1):
        cid = lax.axis_index("c")
        sid = lax.axis_index("s")
        wid = cid * 16 + sid
        ssem = (ssem0, ssem1)
        isem = (isem0, isem1)

        pltpu.sync_copy(b2_hbm.at[0], h2a_l)
        pltpu.sync_copy(b2_hbm.at[1], h2b_l)
        pltpu.sync_copy(b2_hbm.at[2], as2_l)
        pltpu.sync_copy(b2_hbm.at[3], ad2_l)

        @pl.loop(0, npt, step=LANES)
        def _(i):
            zbuf[pl.ds(i, LANES)] = jnp.zeros((LANES,), jnp.float32)

        sl_out = pl.ds(sid * npt, npt)
        pltpu.sync_copy(zbuf, ua_sh.at[sl_out])
        pltpu.sync_copy(zbuf, ub_sh.at[sl_out])
        pltpu.sync_copy(zbuf, d2_sh.at[sl_out])
        plsc.subcore_barrier()

        blk0 = wid * nb

        pltpu.async_copy(e_hbm.at[blk0], eidx3.at[0], isem0)

        @pl.loop(0, nb, step=6)
        def _(b):
            for u in range(6):
                bb = b + u
                x2 = u % 2
                x3 = u % 3
                sidx = eidx3.at[x3, 0]
                didx = eidx3.at[x3, 1]
                didx_o = eidx3.at[(u + 1) % 3, 1]
                pa = pa2.at[x2]
                pb = pb2.at[x2]
                pc = pc2.at[x2]

                def _drain():
                    pltpu.make_async_copy(pa, ua_sh.at[didx_o],
                                          ssem[x2]).wait()
                    pltpu.make_async_copy(pb, ub_sh.at[didx_o],
                                          ssem[x2]).wait()
                    pltpu.make_async_copy(pc, d2_sh.at[didx_o],
                                          ssem[x2]).wait()
                if u < 2:
                    @pl.when(bb >= 2)
                    def _():
                        _drain()
                else:
                    _drain()

                def _pref():
                    pltpu.async_copy(e_hbm.at[blk0 + bb + 1],
                                     eidx3.at[(u + 1) % 3],
                                     isem[(u + 1) % 2])
                if u == 5:
                    @pl.when(bb + 1 < nb)
                    def _():
                        _pref()
                else:
                    _pref()

                pltpu.make_async_copy(e_hbm.at[blk0 + bb], eidx3.at[x3],
                                      isem[x2]).wait()

                for c in range(nchunk):
                    sl = pl.ds(c * LANES, LANES)
                    sv = sidx[sl]
                    dv = didx[sl]
                    e = (plsc.load_gather(as2_l, [sv])
                         + plsc.load_gather(ad2_l, [dv]))
                    e = jnp.where(e >= 0.0, e, 0.2 * e)
                    p = jnp.exp(e)
                    pa[sl] = p * plsc.load_gather(h2a_l, [sv])
                    pb[sl] = p * plsc.load_gather(h2b_l, [sv])
                    pc[sl] = p
                pltpu.async_copy(pa, ua_sh.at[didx], ssem[x2], add=True)
                pltpu.async_copy(pb, ub_sh.at[didx], ssem[x2], add=True)
                pltpu.async_copy(pc, d2_sh.at[didx], ssem[x2], add=True)

        for x2, x3 in ((0, 1), (1, 2)):
            pltpu.make_async_copy(pa2.at[x2], ua_sh.at[eidx3.at[x3, 1]],
                                  ssem[x2]).wait()
            pltpu.make_async_copy(pb2.at[x2], ub_sh.at[eidx3.at[x3, 1]],
                                  ssem[x2]).wait()
            pltpu.make_async_copy(pc2.at[x2], d2_sh.at[eidx3.at[x3, 1]],
                                  ssem[x2]).wait()
        plsc.subcore_barrier()
        pltpu.sync_copy(ua_sh.at[sl_out], o_hbm.at[cid, 0, sl_out])
        pltpu.sync_copy(ub_sh.at[sl_out], o_hbm.at[cid, 1, sl_out])
        pltpu.sync_copy(d2_sh.at[sl_out], o_hbm.at[cid, 2, sl_out])

    return k(b2arr, edges)


def _tc3(o2p, b2, n):
    _, _, np_ = o2p.shape

    def body(o_ref, b2_ref, out_ref):
        u0 = o_ref[0, 0:1, :] + o_ref[1, 0:1, :]
        u1 = o_ref[0, 1:2, :] + o_ref[1, 1:2, :]
        dd = o_ref[0, 2:3, :] + o_ref[1, 2:3, :] + 1e-16
        o0 = u0 / dd + b2_ref[0]
        o1 = u1 / dd + b2_ref[1]
        out_ref[...] = jnp.transpose(jnp.concatenate([o0, o1], 0), (1, 0))

    return pl.pallas_call(
        body,
        grid=(np_ // BR,),
        in_specs=[
            pl.BlockSpec((2, 8, BR), lambda i: (0, 0, i)),
            pl.BlockSpec(memory_space=pltpu.SMEM),
        ],
        out_specs=pl.BlockSpec((BR, 2), lambda i: (i, 0)),
        out_shape=jax.ShapeDtypeStruct((n, 2), jnp.float32),
    )(o2p, b2)


def kernel(x, edge_index, W1, att_src1, att_dst1, b1, W2, att_src2,
           att_dst2, b2):
    n, d = x.shape
    e = edge_index.shape[1]
    n_pad = _round_up(n, BR)
    e_tot = e + n
    nb = _round_up(e_tot, NW * BLK) // (NW * BLK)
    nb = _round_up(nb, 6)
    e_pad = NW * nb * BLK
    n_fill = e_pad - e_tot

    ei = edge_index.astype(jnp.int32)
    loop_idx = jnp.arange(n, dtype=jnp.int32)
    fill = jnp.arange(n_fill, dtype=jnp.int32)
    pad_src = (fill * 131) % n
    pad_dst = n + fill % (n_pad - n)
    src = jnp.concatenate([ei[0], loop_idx, pad_src])
    dst = jnp.concatenate([ei[1], loop_idx, pad_dst])
    edges = jnp.stack([src.reshape(-1, BLK), dst.reshape(-1, BLK)], axis=1)

    x_pad = jnp.pad(x, ((0, n_pad - n), (0, 0)))
    att12 = jnp.stack([att_src1, att_dst1])
    b1r = b1.reshape(1, d)
    w2t = jnp.transpose(W2)

    h, a1 = _tc1(x_pad, W1, att12, n)
    u1p, d1p = _sc1(h, a1, edges, nb)
    b2arr = _tc2(u1p, d1p, b1r, w2t, att_src2, att_dst2, n)
    o2p = _sc2(b2arr, edges, nb)
    return _tc3(o2p, b2, n)
